# Initial kernel scaffold; baseline (speedup 1.0000x reference)
import jax
import jax.numpy as jnp
from jax.experimental import pallas as pl


def kernel(x, W):
    raise NotImplementedError("write your pallas kernel here")


# speedup vs baseline: 4.0324x; 4.0324x over previous
import functools

import numpy as np
import jax
import jax.numpy as jnp
from jax import lax
from jax.experimental import pallas as pl
from jax.experimental.pallas import tpu as pltpu
from jax.experimental.pallas import tpu_sc as plsc

_F = 26
_TOTAL = 26 * 4000
_D = 16
_B = 4096
_OFFS = np.arange(_F, dtype=np.int32) * 4000
_II, _JJ = np.triu_indices(_F, k=1)
_P = _II.shape[0]

_NC = 2
_NS = 16
_NW = _NC * _NS

_ROWS = _B * _P
_SUB = 128
_NSUB = _ROWS // _SUB
_CSUB = 8
_CHUNK = _CSUB * _SUB
_NCHUNK = _NSUB // _CSUB
_K_PER_W = (_NCHUNK + _NW - 1) // _NW

_mesh = plsc.VectorSubcoreMesh(core_axis_name="c", subcore_axis_name="s")


@functools.partial(
    pl.kernel,
    mesh=_mesh,
    out_type=jax.ShapeDtypeStruct((_ROWS, _D), jnp.float32),
    compiler_params=pltpu.CompilerParams(use_tc_tiling_on_sc=False),
    scratch_types=[
        pltpu.VMEM((_CSUB, _SUB), jnp.int32),
        pltpu.VMEM((_CSUB, _SUB), jnp.int32),
        pltpu.VMEM((_CHUNK, _D), jnp.float32),
        pltpu.VMEM((_CHUNK, _D), jnp.float32),
        pltpu.SemaphoreType.DMA,
        pltpu.SemaphoreType.DMA,
    ],
)
def _ffm_sc(wf_hbm, ia_hbm, ib_hbm, out_hbm, ia_v, ib_v, ra_v, rb_v, sa, sb):
    wid = lax.axis_index("s") * _NC + lax.axis_index("c")

    def chunk_body(k, carry):
        g = k * _NW + wid

        @pl.when(g < _NCHUNK)
        def _():
            srow = g * _CSUB
            pltpu.sync_copy(ia_hbm.at[pl.ds(srow, _CSUB)], ia_v)
            pltpu.sync_copy(ib_hbm.at[pl.ds(srow, _CSUB)], ib_v)

            def gather_body(j, c):
                ca = pltpu.async_copy(
                    wf_hbm.at[ia_v.at[j]], ra_v.at[pl.ds(j * _SUB, _SUB)], sa)
                cb = pltpu.async_copy(
                    wf_hbm.at[ib_v.at[j]], rb_v.at[pl.ds(j * _SUB, _SUB)], sb)
                ca.wait()
                cb.wait()
                return c

            lax.fori_loop(0, _CSUB, gather_body, 0)

            def mul_body(i, c):
                ra_v[i, :] = ra_v[i, :] * rb_v[i, :]
                return c

            lax.fori_loop(0, _CHUNK, mul_body, 0)
            pltpu.sync_copy(ra_v, out_hbm.at[pl.ds(srow * _SUB, _CHUNK)])

        return carry

    lax.fori_loop(0, _K_PER_W, chunk_body, 0)


def kernel(x, W):
    offs = jnp.asarray(_OFFS)
    ii = jnp.asarray(_II)
    jj = jnp.asarray(_JJ)
    xi = x + offs[None, :]
    ia = (jj * _TOTAL)[None, :] + jnp.take(xi, ii, axis=1)
    ib = (ii * _TOTAL)[None, :] + jnp.take(xi, jj, axis=1)
    ia = ia.astype(jnp.int32).reshape(_NSUB, _SUB)
    ib = ib.astype(jnp.int32).reshape(_NSUB, _SUB)
    wf = W.reshape(_F * _TOTAL, _D)
    out = _ffm_sc(wf, ia, ib)
    return out.reshape(_B, _P, _D)


# speedup vs baseline: 21.7111x; 5.3842x over previous
import functools

import jax
import jax.numpy as jnp
from jax import lax
from jax.experimental import pallas as pl
from jax.experimental.pallas import tpu as pltpu
from jax.experimental.pallas import tpu_sc as plsc

_F = 26
_V = 4000
_D = 16
_B = 4096
_P = _F * (_F - 1) // 2

_NC = 2
_NS = 16
_NW = _NC * _NS

_DH = _D // 2
_NT = _P * 2

_mesh = plsc.VectorSubcoreMesh(core_axis_name="c", subcore_axis_name="s")


@functools.partial(
    pl.kernel,
    mesh=_mesh,
    out_type=jax.ShapeDtypeStruct((_P, _D, _B), jnp.float32),
    compiler_params=pltpu.CompilerParams(
        use_tc_tiling_on_sc=False, needs_layout_passes=False),
    scratch_types=[
        pltpu.VMEM((_DH, _V), jnp.float32),
        pltpu.VMEM((_DH, _V), jnp.float32),
        pltpu.VMEM((_DH, _B), jnp.float32),
        pltpu.VMEM((1, _B), jnp.int32),
        pltpu.VMEM((1, _B), jnp.int32),
    ],
)
def _ffm_sc(wt_hbm, xc_hbm, out_hbm, sa_v, sb_v, out_v, xa_v, xb_v):
    wid = lax.axis_index("s") * _NC + lax.axis_index("c")

    def do_task(i, j, p, h):
        pltpu.sync_copy(wt_hbm.at[j, pl.ds(h * _DH, _DH), pl.ds(i * _V, _V)],
                        sa_v)
        pltpu.sync_copy(wt_hbm.at[i, pl.ds(h * _DH, _DH), pl.ds(j * _V, _V)],
                        sb_v)
        pltpu.sync_copy(xc_hbm.at[i], xa_v)
        pltpu.sync_copy(xc_hbm.at[j], xb_v)

        def block_body(b0, c):
            xa = xa_v[0, pl.ds(b0 * 16, 16)]
            xb = xb_v[0, pl.ds(b0 * 16, 16)]
            for d in range(_DH):
                row = jnp.full((16,), d, jnp.int32)
                va = plsc.load_gather(sa_v, [row, xa])
                vb = plsc.load_gather(sb_v, [row, xb])
                out_v[d, pl.ds(b0 * 16, 16)] = va * vb
            return c

        lax.fori_loop(0, _B // 16, block_body, 0)
        pltpu.sync_copy(out_v, out_hbm.at[p, pl.ds(h * _DH, _DH)])

    def body_j(j, carry):
        i, p = carry

        for h in range(2):
            t = 2 * p + h

            @pl.when(lax.rem(t, _NW) == wid)
            def _():
                do_task(i, j, p, h)

        return (i, p + 1)

    def body_i(i, p):
        _, p = lax.fori_loop(i + 1, _F, body_j, (i, p))
        return p

    lax.fori_loop(0, _F, body_i, 0)


def kernel(x, W):
    wt = W.transpose(0, 2, 1)
    xc = x.T.reshape(_F, 1, _B)
    out_t = _ffm_sc(wt, xc)
    return out_t.transpose(2, 0, 1)


# speedup vs baseline: 28.2722x; 1.3022x over previous
import functools

import jax
import jax.numpy as jnp
from jax import lax
from jax.experimental import pallas as pl
from jax.experimental.pallas import tpu as pltpu
from jax.experimental.pallas import tpu_sc as plsc

_F = 26
_V = 4000
_D = 16
_B = 4096
_P = _F * (_F - 1) // 2

_NC = 2
_NS = 16
_NW = _NC * _NS

_DQ = 4
_NT = _P * 4
_TPW = _NT // _NW
_XTRA = _NT - _TPW * _NW

_mesh = plsc.VectorSubcoreMesh(core_axis_name="c", subcore_axis_name="s")


@functools.partial(
    pl.kernel,
    mesh=_mesh,
    out_type=jax.ShapeDtypeStruct((_P * _D, 1, _B), jnp.float32),
    compiler_params=pltpu.CompilerParams(
        use_tc_tiling_on_sc=False, needs_layout_passes=False),
    scratch_types=[
        pltpu.VMEM((2, _DQ, 1, _V), jnp.float32),
        pltpu.VMEM((2, _DQ, 1, _V), jnp.float32),
        pltpu.VMEM((2, _DQ, 1, _B), jnp.float32),
        pltpu.VMEM((2, 1, _B), jnp.int32),
        pltpu.VMEM((2, 1, _B), jnp.int32),
        pltpu.SemaphoreType.DMA,
        pltpu.SemaphoreType.DMA,
        pltpu.SemaphoreType.DMA,
        pltpu.SemaphoreType.DMA,
    ],
)
def _ffm_sc(wt_hbm, xc_hbm, out_hbm, sa_v, sb_v, out_v, xa_v, xb_v,
            ld0, ld1, st0, st1):
    wid = lax.axis_index("s") * _NC + lax.axis_index("c")
    start = wid * _TPW + jnp.minimum(wid, _XTRA)
    cnt = jnp.where(wid < _XTRA, _TPW + 1, _TPW)
    end = start + cnt

    lds = (ld0, ld1)
    sts = (st0, st1)

    def unpack(t):
        p = t // 4
        q = t - p * 4

        def bi(_, c):
            i0, rem = c
            n = (_F - 1) - i0
            take = rem >= n
            return (jnp.where(take, i0 + 1, i0), jnp.where(take, rem - n, rem))

        i0, rem = lax.fori_loop(0, _F, bi, (jnp.int32(0), p))
        return (i0, i0 + 1 + rem, p, q)

    def adv(s):
        i, j, p, q = s
        q1 = q + 1
        qw = q1 == _DQ
        q1 = jnp.where(qw, 0, q1)
        p1 = jnp.where(qw, p + 1, p)
        j1 = jnp.where(qw, j + 1, j)
        jw = qw & (j1 == _F)
        i1 = jnp.where(jw, i + 1, i)
        j1 = jnp.where(jw, i1 + 1, j1)
        return (i1, j1, p1, q1)

    def load_descs(s, par):
        i, j, _, q = s
        return (
            pltpu.make_async_copy(
                wt_hbm.at[pl.ds(j * _D + q * _DQ, _DQ), :, pl.ds(i * _V, _V)],
                sa_v.at[par], lds[par]),
            pltpu.make_async_copy(
                wt_hbm.at[pl.ds(i * _D + q * _DQ, _DQ), :, pl.ds(j * _V, _V)],
                sb_v.at[par], lds[par]),
            pltpu.make_async_copy(xc_hbm.at[i], xa_v.at[par], lds[par]),
            pltpu.make_async_copy(xc_hbm.at[j], xb_v.at[par], lds[par]),
        )

    def issue_loads(s, par):
        for d in load_descs(s, par):
            d.start()

    def wait_loads(s, par):
        for d in load_descs(s, par):
            d.wait()

    def store_desc(s, par):
        _, _, p, q = s
        return pltpu.make_async_copy(
            out_v.at[par], out_hbm.at[pl.ds(p * _D + q * _DQ, _DQ)], sts[par])

    def compute(s, par):
        xa_r = xa_v.at[par]
        xb_r = xb_v.at[par]
        sa_r = sa_v.at[par]
        sb_r = sb_v.at[par]
        zero16 = jnp.zeros((16,), jnp.int32)
        rows = [jnp.full((16,), d, jnp.int32) for d in range(_DQ)]

        def block_body(b0, c):
            xa = xa_r[0, pl.ds(b0 * 16, 16)]
            xb = xb_r[0, pl.ds(b0 * 16, 16)]
            for d in range(_DQ):
                va = plsc.load_gather(sa_r, [rows[d], zero16, xa])
                vb = plsc.load_gather(sb_r, [rows[d], zero16, xb])
                out_v[par, d, 0, pl.ds(b0 * 16, 16)] = va * vb
            return c

        lax.fori_loop(0, _B // 16, block_body, 0)

    def phase(t, m, s_cur, s_nxt, par):
        @pl.when(t < end)
        def _():
            @pl.when(t + 1 < end)
            def _():
                issue_loads(s_nxt, 1 - par)

            wait_loads(s_cur, par)

            @pl.when(m > 0)
            def _():
                store_desc(s_cur, par).wait()

            compute(s_cur, par)
            store_desc(s_cur, par).start()

    s0 = unpack(start)
    issue_loads(s0, 0)

    def step(m, carry):
        s0 = carry
        s1 = adv(s0)
        s2 = adv(s1)
        t = start + 2 * m
        phase(t, m, s0, s1, 0)
        phase(t + 1, m, s1, s2, 1)
        return s2

    lax.fori_loop(0, (_TPW + 2) // 2, step, s0)
    for par in range(2):
        pltpu.make_async_copy(
            out_v.at[par], out_hbm.at[pl.ds(0, _DQ)], sts[par]).wait()


def kernel(x, W):
    wt = W.transpose(0, 2, 1).reshape(_F * _D, 1, _V * _F)
    xc = x.T.reshape(_F, 1, _B)
    out3 = _ffm_sc(wt, xc)
    out_t = out3.reshape(_P, _D, _B)
    return out_t.transpose(2, 0, 1)


# speedup vs baseline: 38.9970x; 1.3793x over previous
import functools

import jax
import jax.numpy as jnp
from jax import lax
from jax.experimental import pallas as pl
from jax.experimental.pallas import tpu as pltpu
from jax.experimental.pallas import tpu_sc as plsc

_F = 26
_V = 4000
_D = 16
_B = 4096
_P = _F * (_F - 1) // 2

_NC = 2
_NS = 16
_NW = _NC * _NS

_DQ = 4
_NT = _P * 4
_TPW = _NT // _NW
_XTRA = _NT - _TPW * _NW

_mesh = plsc.VectorSubcoreMesh(core_axis_name="c", subcore_axis_name="s")


@functools.partial(
    pl.kernel,
    mesh=_mesh,
    out_type=jax.ShapeDtypeStruct((_P * _D, 1, _B), jnp.float32),
    compiler_params=pltpu.CompilerParams(
        use_tc_tiling_on_sc=False, needs_layout_passes=False),
    scratch_types=[
        pltpu.VMEM((2, _DQ, 1, _V), jnp.float32),
        pltpu.VMEM((2, _DQ, 1, _V), jnp.float32),
        pltpu.VMEM((2, _DQ, 1, _B), jnp.float32),
        pltpu.VMEM((2, 1, _B), jnp.int32),
        pltpu.VMEM((2, 1, _B), jnp.int32),
        pltpu.SemaphoreType.DMA,
        pltpu.SemaphoreType.DMA,
        pltpu.SemaphoreType.DMA,
        pltpu.SemaphoreType.DMA,
    ],
)
def _ffm_sc(wt_hbm, xc_hbm, out_hbm, sa_v, sb_v, out_v, xa_v, xb_v,
            ld0, ld1, st0, st1):
    wid = lax.axis_index("s") * _NC + lax.axis_index("c")
    start = wid * _TPW + jnp.minimum(wid, _XTRA)
    cnt = jnp.where(wid < _XTRA, _TPW + 1, _TPW)
    end = start + cnt

    lds = (ld0, ld1)
    sts = (st0, st1)

    def unpack(t):
        p = t // 4
        q = t - p * 4

        def bi(_, c):
            i0, rem = c
            n = (_F - 1) - i0
            take = rem >= n
            return (jnp.where(take, i0 + 1, i0), jnp.where(take, rem - n, rem))

        i0, rem = lax.fori_loop(0, _F, bi, (jnp.int32(0), p))
        return (i0, i0 + 1 + rem, p, q)

    def adv(s):
        i, j, p, q = s
        q1 = q + 1
        qw = q1 == _DQ
        q1 = jnp.where(qw, 0, q1)
        p1 = jnp.where(qw, p + 1, p)
        j1 = jnp.where(qw, j + 1, j)
        jw = qw & (j1 == _F)
        i1 = jnp.where(jw, i + 1, i)
        j1 = jnp.where(jw, i1 + 1, j1)
        return (i1, j1, p1, q1)

    def load_descs(s, par):
        i, j, _, q = s
        return (
            pltpu.make_async_copy(
                wt_hbm.at[pl.ds(j * _D + q * _DQ, _DQ), :, pl.ds(i * _V, _V)],
                sa_v.at[par], lds[par]),
            pltpu.make_async_copy(
                wt_hbm.at[pl.ds(i * _D + q * _DQ, _DQ), :, pl.ds(j * _V, _V)],
                sb_v.at[par], lds[par]),
            pltpu.make_async_copy(xc_hbm.at[i], xa_v.at[par], lds[par]),
            pltpu.make_async_copy(xc_hbm.at[j], xb_v.at[par], lds[par]),
        )

    def issue_loads(s, par):
        for d in load_descs(s, par):
            d.start()

    def wait_loads(s, par):
        for d in load_descs(s, par):
            d.wait()

    def store_desc(s, par):
        _, _, p, q = s
        return pltpu.make_async_copy(
            out_v.at[par], out_hbm.at[pl.ds(p * _D + q * _DQ, _DQ)], sts[par])

    def compute(s, par):
        xa_r = xa_v.at[par]
        xb_r = xb_v.at[par]
        sa_d = [sa_v.at[par, d, 0] for d in range(_DQ)]
        sb_d = [sb_v.at[par, d, 0] for d in range(_DQ)]

        @plsc.parallel_loop(0, _B // 16, unroll=4)
        def block_body(b0):
            xa = xa_r[0, pl.ds(b0 * 16, 16)]
            xb = xb_r[0, pl.ds(b0 * 16, 16)]
            for d in range(_DQ):
                va = plsc.load_gather(sa_d[d], [xa])
                vb = plsc.load_gather(sb_d[d], [xb])
                out_v[par, d, 0, pl.ds(b0 * 16, 16)] = va * vb

    def phase(t, m, s_cur, s_nxt, par):
        @pl.when(t < end)
        def _():
            @pl.when(t + 1 < end)
            def _():
                issue_loads(s_nxt, 1 - par)

            wait_loads(s_cur, par)

            @pl.when(m > 0)
            def _():
                store_desc(s_cur, par).wait()

            compute(s_cur, par)
            store_desc(s_cur, par).start()

    s0 = unpack(start)
    issue_loads(s0, 0)

    def step(m, carry):
        s0 = carry
        s1 = adv(s0)
        s2 = adv(s1)
        t = start + 2 * m
        phase(t, m, s0, s1, 0)
        phase(t + 1, m, s1, s2, 1)
        return s2

    lax.fori_loop(0, (_TPW + 2) // 2, step, s0)
    for par in range(2):
        pltpu.make_async_copy(
            out_v.at[par], out_hbm.at[pl.ds(0, _DQ)], sts[par]).wait()


def kernel(x, W):
    wt = W.transpose(0, 2, 1).reshape(_F * _D, 1, _V * _F)
    xc = x.T.reshape(_F, 1, _B)
    out3 = _ffm_sc(wt, xc)
    out_t = out3.reshape(_P, _D, _B)
    return out_t.transpose(2, 0, 1)
